# baseline (speedup 1.0000x reference)
import functools

import jax
import jax.numpy as jnp
from jax import lax
from jax.experimental import pallas as pl
from jax.experimental.pallas import tpu as pltpu
from jax.experimental.pallas import tpu_sc as plsc

BATCH = 16384
EMB_DIM = 64
LANES = 16

try:
    _info = plsc.get_sparse_core_info()
    NUM_CORES, NUM_SUBCORES = _info.num_cores, _info.num_subcores
except Exception:
    NUM_CORES, NUM_SUBCORES = 2, 16

NUM_WORKERS = NUM_CORES * NUM_SUBCORES
ROWS_PER_WORKER = BATCH // NUM_WORKERS
CHUNK = 128
CHUNKS_PER_WORKER = ROWS_PER_WORKER // CHUNK
IDX_COLS = CHUNK
IDX_ROWS = BATCH // IDX_COLS


def _body(heads_hbm, rels_hbm, tails_hbm, ent_hbm, rel_hbm, out_hbm,
          idx_h, idx_r, idx_t, rows_h, rows_r, rows_t, out_v, tr_v, sem):
    wid = lax.axis_index("s") * NUM_CORES + lax.axis_index("c")
    idx_base = wid * CHUNKS_PER_WORKER
    row_base = wid * ROWS_PER_WORKER

    pltpu.sync_copy(heads_hbm.at[pl.ds(idx_base, CHUNKS_PER_WORKER)], idx_h)
    pltpu.sync_copy(rels_hbm.at[pl.ds(idx_base, CHUNKS_PER_WORKER)], idx_r)
    pltpu.sync_copy(tails_hbm.at[pl.ds(idx_base, CHUNKS_PER_WORKER)], idx_t)

    copies = []
    for j in range(CHUNKS_PER_WORKER):
        dst = pl.ds(j * CHUNK, CHUNK)
        copies.append(pltpu.async_copy(ent_hbm.at[idx_h.at[j]], rows_h.at[dst], sem))
        copies.append(pltpu.async_copy(rel_hbm.at[idx_r.at[j]], rows_r.at[dst], sem))
        copies.append(pltpu.async_copy(ent_hbm.at[idx_t.at[j]], rows_t.at[dst], sem))
    for c in copies:
        c.wait()

    lane = lax.iota(jnp.int32, LANES)
    bfly = [(lane ^ m).reshape(LANES, 1) for m in (1, 2, 4, 8)]
    _gdn = lax.GatherDimensionNumbers(
        offset_dims=(), collapsed_slice_dims=(0,), start_index_map=(0,))

    def _shuffle(x, idx):
        return lax.gather(x, idx, _gdn, (1,),
                          mode=lax.GatherScatterMode.PROMISE_IN_BOUNDS)

    def group_step(g, carry):
        base = g * LANES
        vec = jnp.zeros((LANES,), jnp.float32)
        for j in range(LANES):
            i = base + j
            acc = None
            for k in range(EMB_DIM // LANES):
                sl = pl.ds(k * LANES, LANES)
                d = rows_h[i, sl] + rows_r[i, sl] - rows_t[i, sl]
                sq = d * d
                acc = sq if acc is None else acc + sq
            for m in bfly:
                acc = acc + _shuffle(acc, m)
            vec = jnp.where(lane == j, acc, vec)
        out_v[pl.ds(base, LANES)] = vec
        return carry

    lax.fori_loop(0, ROWS_PER_WORKER // LANES, group_step, 0)

    pltpu.sync_copy(out_v, out_hbm.at[pl.ds(row_base, ROWS_PER_WORKER)])


@functools.partial(jax.jit, static_argnums=())
def _transe_sc(heads2, rels2, tails2, ent_embeddings, rel_embeddings):
    mesh = plsc.VectorSubcoreMesh(core_axis_name="c", subcore_axis_name="s")
    return pl.kernel(
        _body,
        mesh=mesh,
        compiler_params=pltpu.CompilerParams(use_tc_tiling_on_sc=False),
        out_type=jax.ShapeDtypeStruct((BATCH,), jnp.float32),
        scratch_types=[
            pltpu.VMEM((CHUNKS_PER_WORKER, IDX_COLS), jnp.int32),
            pltpu.VMEM((CHUNKS_PER_WORKER, IDX_COLS), jnp.int32),
            pltpu.VMEM((CHUNKS_PER_WORKER, IDX_COLS), jnp.int32),
            pltpu.VMEM((ROWS_PER_WORKER, EMB_DIM), jnp.float32),
            pltpu.VMEM((ROWS_PER_WORKER, EMB_DIM), jnp.float32),
            pltpu.VMEM((ROWS_PER_WORKER, EMB_DIM), jnp.float32),
            pltpu.VMEM((ROWS_PER_WORKER,), jnp.float32),
            pltpu.VMEM((LANES * (LANES + 1),), jnp.float32),
            pltpu.SemaphoreType.DMA,
        ],
    )(heads2, rels2, tails2, ent_embeddings, rel_embeddings)


def kernel(heads, relations, tails, ent_embeddings, rel_embeddings):
    heads2 = heads.astype(jnp.int32).reshape(IDX_ROWS, IDX_COLS)
    rels2 = relations.astype(jnp.int32).reshape(IDX_ROWS, IDX_COLS)
    tails2 = tails.astype(jnp.int32).reshape(IDX_ROWS, IDX_COLS)
    return _transe_sc(heads2, rels2, tails2, ent_embeddings, rel_embeddings)


# speedup vs baseline: 1.3950x; 1.3950x over previous
import functools

import jax
import jax.numpy as jnp
from jax import lax
from jax.experimental import pallas as pl
from jax.experimental.pallas import tpu as pltpu
from jax.experimental.pallas import tpu_sc as plsc

BATCH = 16384
EMB_DIM = 64
LANES = 16
GRP = 8

try:
    _info = plsc.get_sparse_core_info()
    NUM_CORES, NUM_SUBCORES = _info.num_cores, _info.num_subcores
except Exception:
    NUM_CORES, NUM_SUBCORES = 2, 16

NUM_WORKERS = NUM_CORES * NUM_SUBCORES
ROWS_PER_WORKER = BATCH // NUM_WORKERS
NGROUPS = ROWS_PER_WORKER // LANES


def _body(heads_hbm, rels_hbm, tails_hbm, ent_hbm, rel_hbm, out_hbm,
          idx_h, idx_r, idx_t, h0, h1, r0, r1, t0, t1, out_v, sem0, sem1):
    wid = lax.axis_index("s") * NUM_CORES + lax.axis_index("c")
    base = wid * ROWS_PER_WORKER

    pltpu.sync_copy(heads_hbm.at[pl.ds(base, ROWS_PER_WORKER)], idx_h)
    pltpu.sync_copy(rels_hbm.at[pl.ds(base, ROWS_PER_WORKER)], idx_r)
    pltpu.sync_copy(tails_hbm.at[pl.ds(base, ROWS_PER_WORKER)], idx_t)

    hbuf = (h0, h1)
    rbuf = (r0, r1)
    tbuf = (t0, t1)
    sems = (sem0, sem1)

    lane = lax.iota(jnp.int32, LANES)
    bfly = [(lane ^ m).reshape(LANES, 1) for m in (1, 2, 4, 8)]
    _gdn = lax.GatherDimensionNumbers(
        offset_dims=(), collapsed_slice_dims=(0,), start_index_map=(0,))

    def _shuffle(x, idx):
        return lax.gather(x, idx, _gdn, (1,),
                          mode=lax.GatherScatterMode.PROMISE_IN_BOUNDS)

    def fire(g, s):
        gsl = pl.ds(g * LANES, LANES)
        grp_h = (idx_h[gsl] >> 3) * GRP
        grp_r = (idx_r[gsl] >> 3) * GRP
        grp_t = (idx_t[gsl] >> 3) * GRP
        cps = []
        for j in range(LANES):
            gh = pl.multiple_of(grp_h[j], GRP)
            gr = pl.multiple_of(grp_r[j], GRP)
            gt = pl.multiple_of(grp_t[j], GRP)
            dst = pl.ds(j * GRP, GRP)
            cps.append(pltpu.async_copy(
                ent_hbm.at[pl.ds(gh, GRP)], hbuf[s].at[dst], sems[s]))
            cps.append(pltpu.async_copy(
                rel_hbm.at[pl.ds(gr, GRP)], rbuf[s].at[dst], sems[s]))
            cps.append(pltpu.async_copy(
                ent_hbm.at[pl.ds(gt, GRP)], tbuf[s].at[dst], sems[s]))
        return cps

    def compute(g, s):
        hb, rb, tb = hbuf[s], rbuf[s], tbuf[s]
        gsl = pl.ds(g * LANES, LANES)
        sub_h = idx_h[gsl] & (GRP - 1)
        sub_r = idx_r[gsl] & (GRP - 1)
        sub_t = idx_t[gsl] & (GRP - 1)
        vec = jnp.zeros((LANES,), jnp.float32)
        for j in range(LANES):
            rh = j * GRP + sub_h[j]
            rr = j * GRP + sub_r[j]
            rt = j * GRP + sub_t[j]
            acc = None
            for k in range(EMB_DIM // LANES):
                sl = pl.ds(k * LANES, LANES)
                d = hb[rh, sl] + rb[rr, sl] - tb[rt, sl]
                sq = d * d
                acc = sq if acc is None else acc + sq
            for m in bfly:
                acc = acc + _shuffle(acc, m)
            vec = jnp.where(lane == j, acc, vec)
        out_v[pl.ds(g * LANES, LANES)] = vec

    def group_pair(p, carry):
        g0 = p * 2
        for cp in fire(g0, 0):
            cp.wait()
        cps1 = fire(g0 + 1, 1)
        compute(g0, 0)
        for cp in cps1:
            cp.wait()
        compute(g0 + 1, 1)
        return carry

    lax.fori_loop(0, NGROUPS // 2, group_pair, 0)

    pltpu.sync_copy(out_v, out_hbm.at[pl.ds(base, ROWS_PER_WORKER)])


@functools.partial(jax.jit, static_argnums=())
def _transe_sc(heads, rels, tails, ent, rel):
    mesh = plsc.VectorSubcoreMesh(core_axis_name="c", subcore_axis_name="s")
    return pl.kernel(
        _body,
        mesh=mesh,
        compiler_params=pltpu.CompilerParams(use_tc_tiling_on_sc=True),
        out_type=jax.ShapeDtypeStruct((BATCH,), jnp.float32),
        scratch_types=[
            pltpu.VMEM((ROWS_PER_WORKER,), jnp.int32),
            pltpu.VMEM((ROWS_PER_WORKER,), jnp.int32),
            pltpu.VMEM((ROWS_PER_WORKER,), jnp.int32),
            pltpu.VMEM((LANES * GRP, EMB_DIM), jnp.float32),
            pltpu.VMEM((LANES * GRP, EMB_DIM), jnp.float32),
            pltpu.VMEM((LANES * GRP, EMB_DIM), jnp.float32),
            pltpu.VMEM((LANES * GRP, EMB_DIM), jnp.float32),
            pltpu.VMEM((LANES * GRP, EMB_DIM), jnp.float32),
            pltpu.VMEM((LANES * GRP, EMB_DIM), jnp.float32),
            pltpu.VMEM((ROWS_PER_WORKER,), jnp.float32),
            pltpu.SemaphoreType.DMA,
            pltpu.SemaphoreType.DMA,
        ],
    )(heads, rels, tails, ent, rel)


def kernel(heads, relations, tails, ent_embeddings, rel_embeddings):
    return _transe_sc(heads.astype(jnp.int32), relations.astype(jnp.int32),
                      tails.astype(jnp.int32), ent_embeddings, rel_embeddings)
